# baseline (speedup 1.0000x reference)
import functools

import jax
import jax.numpy as jnp
from jax import lax
from jax.experimental import pallas as pl
from jax.experimental.pallas import tpu as pltpu
from jax.experimental.pallas import tpu_sc as plsc

BATCH = 4096
HIST_LEN = 20
VOCAB = 1000

NUM_CORES = 2
NUM_SUBCORES = 16
NUM_WORKERS = NUM_CORES * NUM_SUBCORES
ROWS_PER_WORKER = BATCH // NUM_WORKERS
LANES = 16
BLOCKS = ROWS_PER_WORKER // LANES


def _sc_body(xt_hbm, out_hbm, idx_v, hist_v):
    c = lax.axis_index("c")
    s = lax.axis_index("s")
    wid = s * NUM_CORES + c
    row_base = wid * ROWS_PER_WORKER

    pltpu.sync_copy(xt_hbm.at[wid], idx_v)

    lane_off = lax.iota(jnp.int32, LANES) * VOCAB
    ones = jnp.ones((LANES,), jnp.float32)
    zeros = jnp.zeros((LANES,), jnp.float32)

    for k in range(LANES * VOCAB // LANES):
        hist_v[pl.ds(k * LANES, LANES)] = zeros

    for r in range(BLOCKS):
        addrs = []
        for l in range(HIST_LEN):
            col = idx_v[l, pl.ds(r * LANES, LANES)]
            addr = lane_off + col
            addrs.append(addr)
            plsc.addupdate_scatter(hist_v, [addr], ones)
        out_off = (row_base + r * LANES) * VOCAB
        pltpu.sync_copy(hist_v, out_hbm.at[pl.ds(out_off, LANES * VOCAB)])
        for addr in addrs:
            plsc.store_scatter(hist_v, [addr], zeros)


def _make_sc_kernel():
    mesh = plsc.VectorSubcoreMesh(core_axis_name="c", subcore_axis_name="s")
    return functools.partial(
        pl.kernel,
        mesh=mesh,
        out_type=jax.ShapeDtypeStruct((BATCH * VOCAB,), jnp.float32),
        scratch_types=[
            pltpu.VMEM((HIST_LEN, ROWS_PER_WORKER), jnp.int32),
            pltpu.VMEM((LANES * VOCAB,), jnp.float32),
        ],
        compiler_params=pltpu.CompilerParams(needs_layout_passes=False),
    )(_sc_body)


_sc_kernel = _make_sc_kernel()


@jax.jit
def kernel(x):
    xt = x.T.reshape(HIST_LEN, NUM_WORKERS, ROWS_PER_WORKER).transpose(1, 0, 2)
    out_flat = _sc_kernel(xt)
    return out_flat.reshape(BATCH, VOCAB)
